# Initial kernel scaffold; baseline (speedup 1.0000x reference)
import jax
import jax.numpy as jnp
from jax.experimental import pallas as pl


def kernel(x, edge_index, W0_0, W0_1, W0_2, W1_0, W1_1, W1_2, ln0_g, ln0_b, ln1_g, ln1_b, linW, linb):
    raise NotImplementedError("write your pallas kernel here")


# speedup vs baseline: 5.6243x; 5.6243x over previous
import functools

import jax
import jax.numpy as jnp
from jax import lax
from jax.experimental import pallas as pl
from jax.experimental.pallas import tpu as pltpu
from jax.experimental.pallas import tpu_sc as plsc

NC = 2
NS = 16
CH = 128
MROW = 10240
RPT = MROW // NS
BN = 512


def _sc_mesh():
    return plsc.VectorSubcoreMesh(core_axis_name="c", subcore_axis_name="s")


DEGW = 128


def _make_deg_kernel(K):
    @functools.partial(
        pl.kernel,
        out_type=jax.ShapeDtypeStruct((NC, MROW, DEGW), jnp.float32),
        mesh=_sc_mesh(),
        scratch_types=[
            pltpu.VMEM((CH,), jnp.int32),
            pltpu.VMEM((CH, DEGW), jnp.float32),
            pltpu.VMEM((CH, DEGW), jnp.float32),
            pltpu.VMEM_SHARED((MROW, DEGW), jnp.float32),
        ],
    )
    def deg_kernel(dstr, ones_hbm, zeros_hbm, out_hbm, didx, ones_v, buf, acc):
        c = lax.axis_index("c")
        s = lax.axis_index("s")
        pltpu.sync_copy(ones_hbm, ones_v)
        pltpu.sync_copy(zeros_hbm, buf)
        for k in range(RPT // CH):
            pltpu.sync_copy(buf, acc.at[pl.ds(s * RPT + k * CH, CH)])
        plsc.subcore_barrier()

        def body(j, carry):
            pltpu.sync_copy(dstr.at[c, s, j], didx)
            pltpu.sync_copy(ones_v, acc.at[didx], add=True)
            return carry

        lax.fori_loop(0, K, body, 0)
        plsc.subcore_barrier()
        for k in range(RPT // CH):
            r0 = s * RPT + k * CH
            pltpu.sync_copy(acc.at[pl.ds(r0, CH)], buf)
            pltpu.sync_copy(buf, out_hbm.at[c, pl.ds(r0, CH)])

    return deg_kernel


def _make_spass_kernel(K, D):
    @functools.partial(
        pl.kernel,
        out_type=jax.ShapeDtypeStruct((NC, MROW, D), jnp.float32),
        mesh=_sc_mesh(),
        scratch_types=[
            pltpu.VMEM((CH,), jnp.int32),
            pltpu.VMEM((CH,), jnp.int32),
            pltpu.VMEM((CH, D), jnp.float32),
            pltpu.VMEM_SHARED((MROW, D), jnp.float32),
            pltpu.SemaphoreType.DMA,
        ],
    )
    def spass(t_hbm, srcr, dstr, zrows_hbm, out_hbm, sidx, didx, rows, acc, sem):
        c = lax.axis_index("c")
        s = lax.axis_index("s")
        pltpu.sync_copy(zrows_hbm, rows)
        for k in range(RPT // CH):
            pltpu.sync_copy(rows, acc.at[pl.ds(s * RPT + k * CH, CH)])
        plsc.subcore_barrier()

        def body(j, carry):
            pltpu.sync_copy(srcr.at[c, s, j], sidx)
            pltpu.sync_copy(dstr.at[c, s, j], didx)
            pltpu.async_copy(t_hbm.at[sidx], rows, sem).wait()
            pltpu.sync_copy(rows, acc.at[didx], add=True)
            return carry

        lax.fori_loop(0, K, body, 0)
        plsc.subcore_barrier()
        for k in range(RPT // CH):
            r0 = s * RPT + k * CH
            pltpu.sync_copy(acc.at[pl.ds(r0, CH)], rows)
            pltpu.sync_copy(rows, out_hbm.at[c, pl.ds(r0, CH)])

    return spass


def _rowspec(d=128):
    return pl.BlockSpec((BN, d), lambda i: (i, 0))


def _pairspec(d=128):
    return pl.BlockSpec((NC, BN, d), lambda i: (0, i, 0))


def _fullspec(shape):
    n = len(shape)
    return pl.BlockSpec(shape, lambda i, _n=n: (0,) * _n)


def _prep_body(deg_ref, x_ref, dis_ref, t0_ref):
    deg = deg_ref[0, :, 0:1] + deg_ref[1, :, 0:1]
    dis = jnp.where(deg > 0.0, lax.rsqrt(jnp.maximum(deg, 1.0)), 0.0)
    d = jnp.broadcast_to(dis, (BN, 128))
    dis_ref[...] = d
    t0_ref[...] = d * x_ref[...]


def _mid_body(s_ref, dis_ref, u_ref, a_ref):
    ssum = s_ref[0] + s_ref[1]
    d = dis_ref[...]
    a_ref[...] = d * ssum
    u_ref[...] = (d * d) * ssum


def _ln(h, g, b):
    mu = jnp.mean(h, axis=-1, keepdims=True)
    var = jnp.mean((h - mu) ** 2, axis=-1, keepdims=True)
    return (h - mu) * lax.rsqrt(var + 1e-5) * g + b


def _gamma_block(x_ref, a_ref, s_ref, dis_ref, w0_ref, w1_ref, w2_ref, g_ref, b_ref):
    d = dis_ref[...]
    a2 = d * (s_ref[0] + s_ref[1])
    h = jnp.dot(x_ref[...], w0_ref[...], preferred_element_type=jnp.float32)
    h = h + jnp.dot(a_ref[...], w1_ref[...], preferred_element_type=jnp.float32)
    h = h + jnp.dot(a2, w2_ref[...], preferred_element_type=jnp.float32)
    return _ln(h, g_ref[...], b_ref[...])


def _layer_body(x_ref, a_ref, s_ref, dis_ref, w0_ref, w1_ref, w2_ref, g_ref,
                b_ref, h_ref, t_ref):
    hn = _gamma_block(x_ref, a_ref, s_ref, dis_ref, w0_ref, w1_ref, w2_ref,
                      g_ref, b_ref)
    h_ref[...] = hn
    t_ref[...] = dis_ref[...] * hn


def _final_body(x_ref, a_ref, s_ref, dis_ref, w0_ref, w1_ref, w2_ref, g_ref,
                b_ref, lw_ref, lb_ref, out_ref):
    hn = _gamma_block(x_ref, a_ref, s_ref, dis_ref, w0_ref, w1_ref, w2_ref,
                      g_ref, b_ref)
    logits = jnp.dot(hn, lw_ref[...], preferred_element_type=jnp.float32)
    logits = logits + lb_ref[...]
    m = jnp.max(logits, axis=-1, keepdims=True)
    lse = jnp.log(jnp.sum(jnp.exp(logits - m), axis=-1, keepdims=True)) + m
    out_ref[...] = logits - lse


def kernel(x, edge_index, W0_0, W0_1, W0_2, W1_0, W1_1, W1_2, ln0_g, ln0_b,
           ln1_g, ln1_b, linW, linb):
    n, d_in = x.shape
    e = edge_index.shape[1]
    grid = (MROW // BN,)
    k_per_tile = pl.cdiv(e, NC * NS * CH)
    ep = NC * NS * CH * k_per_tile
    pad = ep - e

    src = jnp.concatenate(
        [edge_index[0], jnp.zeros((pad,), jnp.int32)]).reshape(NC, NS, k_per_tile, CH)
    dst_pad = n + (jnp.arange(pad, dtype=jnp.int32) % (MROW - n))
    dst = jnp.concatenate(
        [edge_index[1], dst_pad]).reshape(NC, NS, k_per_tile, CH)
    xp = jnp.pad(x, ((0, MROW - n), (0, 0)))
    ones16 = jnp.ones((CH, DEGW), jnp.float32)
    zeros16 = jnp.zeros((CH, DEGW), jnp.float32)
    zrows = jnp.zeros((CH, d_in), jnp.float32)
    g0 = ln0_g.reshape(1, -1)
    b0 = ln0_b.reshape(1, -1)
    g1 = ln1_g.reshape(1, -1)
    b1 = ln1_b.reshape(1, -1)
    lb = linb.reshape(1, -1)

    deg2 = _make_deg_kernel(k_per_tile)(dst, ones16, zeros16)

    dis, t0 = pl.pallas_call(
        _prep_body,
        grid=grid,
        in_specs=[_pairspec(DEGW), _rowspec()],
        out_specs=[_rowspec(), _rowspec()],
        out_shape=[jax.ShapeDtypeStruct((MROW, 128), jnp.float32)] * 2,
    )(deg2, xp)

    spass = _make_spass_kernel(k_per_tile, d_in)

    def mid(spartial):
        return pl.pallas_call(
            _mid_body,
            grid=grid,
            in_specs=[_pairspec(), _rowspec()],
            out_specs=[_rowspec(), _rowspec()],
            out_shape=[jax.ShapeDtypeStruct((MROW, 128), jnp.float32)] * 2,
        )(spartial, dis)

    s1 = spass(t0, src, dst, zrows)
    u1, a1 = mid(s1)
    s2 = spass(u1, src, dst, zrows)

    h1, t1 = pl.pallas_call(
        _layer_body,
        grid=grid,
        in_specs=[_rowspec(), _rowspec(), _pairspec(), _rowspec(),
                  _fullspec((128, 128)), _fullspec((128, 128)),
                  _fullspec((128, 128)), _fullspec((1, 128)),
                  _fullspec((1, 128))],
        out_specs=[_rowspec(), _rowspec()],
        out_shape=[jax.ShapeDtypeStruct((MROW, 128), jnp.float32)] * 2,
    )(xp, a1, s2, dis, W0_0, W0_1, W0_2, g0, b0)

    s3 = spass(t1, src, dst, zrows)
    u2, a3 = mid(s3)
    s4 = spass(u2, src, dst, zrows)

    out = pl.pallas_call(
        _final_body,
        grid=grid,
        in_specs=[_rowspec(), _rowspec(), _pairspec(), _rowspec(),
                  _fullspec((128, 128)), _fullspec((128, 128)),
                  _fullspec((128, 128)), _fullspec((1, 128)),
                  _fullspec((1, 128)), _fullspec((128, 128)),
                  _fullspec((1, 128))],
        out_specs=_rowspec(),
        out_shape=jax.ShapeDtypeStruct((MROW, 128), jnp.float32),
    )(h1, a3, s4, dis, W1_0, W1_1, W1_2, g1, b1, linW, lb)

    return out[:n]
